# baseline (speedup 1.0000x reference)
import functools

import jax
import jax.numpy as jnp
from jax import lax
from jax.experimental import pallas as pl
from jax.experimental.pallas import tpu as pltpu
from jax.experimental.pallas import tpu_sc as plsc

_N_CLASSES = 100000
_B = 2048
_SMOOTHING = 0.1
_BASE = _SMOOTHING / (_N_CLASSES - 1)
_CONF = 1.0 - _SMOOTHING

_B_TC = 1024
_B_SC = _B - _B_TC

_BB = 32
_NSTREAM = 2
_NBLK = _B_TC // (_BB * _NSTREAM)
_ROWS_PER_STREAM = _B_TC // _NSTREAM


def _tc_gather_rows(t_sref, x_ref, row0, lane):
    gacc = jnp.zeros((1, 1), jnp.float32)
    for r in range(_BB):
        t = t_sref[row0 + r]
        base = (t // 128) * 128
        w = x_ref[pl.ds(r, 1), pl.ds(base, 128)]
        hit = (base + lane) == t
        gacc += jnp.sum(jnp.where(hit, w, 0.0)).reshape(1, 1)
    return gacc


def _tc_body(t_sref, x0_ref, x1_ref, sum_ref, gsum_ref):
    j = pl.program_id(0)

    @pl.when(j == 0)
    def _init():
        sum_ref[...] = jnp.zeros((1, 1), jnp.float32)
        gsum_ref[...] = jnp.zeros((1, 1), jnp.float32)

    sum_ref[...] += (jnp.sum(x0_ref[...]) + jnp.sum(x1_ref[...])).reshape(1, 1)

    lane = lax.broadcasted_iota(jnp.int32, (1, 128), 1)
    gsum_ref[...] += _tc_gather_rows(t_sref, x0_ref, j * _BB, lane)
    gsum_ref[...] += _tc_gather_rows(t_sref, x1_ref,
                                     _ROWS_PER_STREAM + j * _BB, lane)


def _tc_part(tgt, lsm):
    return pl.pallas_call(
        _tc_body,
        grid_spec=pltpu.PrefetchScalarGridSpec(
            num_scalar_prefetch=1,
            grid=(_NBLK,),
            in_specs=[
                pl.BlockSpec((_BB, _N_CLASSES), lambda j, t: (j, 0)),
                pl.BlockSpec((_BB, _N_CLASSES),
                             lambda j, t: (j + _NBLK, 0)),
            ],
            out_specs=[
                pl.BlockSpec((1, 1), lambda j, t: (0, 0)),
                pl.BlockSpec((1, 1), lambda j, t: (0, 0)),
            ],
        ),
        out_shape=[
            jax.ShapeDtypeStruct((1, 1), jnp.float32),
            jax.ShapeDtypeStruct((1, 1), jnp.float32),
        ],
    )(tgt, lsm, lsm)


_NW = 32
_RPT = _B_SC // _NW
_NG = _RPT // 8
_CHC = 1408
_NCH = 99968 // _CHC
_TAIL0 = _NCH * _CHC
_NQ = _NG * _NCH


@functools.partial(
    pl.kernel,
    mesh=plsc.VectorSubcoreMesh(core_axis_name="c", subcore_axis_name="s"),
    out_type=[
        jax.ShapeDtypeStruct((_NW, 16), jnp.float32),
        jax.ShapeDtypeStruct((_NW, 16), jnp.float32),
    ],
    scratch_types=[
        pltpu.VMEM((_RPT,), jnp.int32),
        pltpu.VMEM((16,), jnp.float32),
        pltpu.VMEM((8, _CHC), jnp.float32),
        pltpu.VMEM((8, _CHC), jnp.float32),
        pltpu.VMEM((8, 32), jnp.float32),
        pltpu.VMEM((16,), jnp.float32),
        pltpu.VMEM((16,), jnp.float32),
        pltpu.SemaphoreType.DMA,
        pltpu.SemaphoreType.DMA,
    ],
    compiler_params=pltpu.CompilerParams(use_tc_tiling_on_sc=True),
)
def _sc_part(tgt_hbm, lsm_hbm, dsum_out, gsum_out,
             tgt_v, row_v, buf0, buf1, tail_v, dacc_v, gacc_v, sem0, sem1):
    wid = lax.axis_index("s") * 2 + lax.axis_index("c")
    row0 = _B_TC + wid * _RPT
    pltpu.sync_copy(tgt_hbm.at[pl.ds(row0, _RPT)], tgt_v)

    iota = lax.iota(jnp.int32, 16)
    gacc = jnp.zeros((16,), jnp.float32)
    for j in range(_RPT // 16):
        t_vec = tgt_v[pl.ds(j * 16, 16)]
        for i in range(16):
            t = t_vec[i]
            start = (t // 16) * 16
            b = row0 + j * 16 + i
            pltpu.sync_copy(lsm_hbm.at[b, pl.ds(start, 16)], row_v)
            gacc = gacc + jnp.where(iota == (t - start), row_v[...], 0.0)
    gacc_v[...] = gacc
    pltpu.sync_copy(gacc_v, gsum_out.at[wid])

    bufs = (buf0, buf1)
    sems = (sem0, sem1)

    def _chunk_slice(q):
        b0 = row0 + (q // _NCH) * 8
        c0 = (q % _NCH) * _CHC
        return lsm_hbm.at[pl.ds(b0, 8), pl.ds(c0, _CHC)]

    pltpu.async_copy(_chunk_slice(0), buf0, sem0)
    pltpu.async_copy(_chunk_slice(1), buf1, sem1)

    def _reduce_chunk(buf, acc):
        def inner(i, a):
            for r in range(8):
                a = a + buf[r, pl.ds(i * 16, 16)]
            return a
        return lax.fori_loop(0, _CHC // 16, inner, acc)

    def pair_body(k, dacc):
        for par in range(2):
            q = 2 * k + par
            pltpu.make_async_copy(_chunk_slice(q), bufs[par], sems[par]).wait()
            dacc = _reduce_chunk(bufs[par], dacc)

            @pl.when(q + 2 < _NQ)
            def _issue():
                pltpu.async_copy(_chunk_slice(q + 2), bufs[par], sems[par])
        return dacc

    dacc = lax.fori_loop(0, _NQ // 2, pair_body, jnp.zeros((16,), jnp.float32))

    for g in range(_NG):
        pltpu.sync_copy(
            lsm_hbm.at[pl.ds(row0 + g * 8, 8), pl.ds(_TAIL0, 32)], tail_v)
        for r in range(8):
            dacc = dacc + tail_v[r, pl.ds(0, 16)] + tail_v[r, pl.ds(16, 16)]

    dacc_v[...] = dacc
    pltpu.sync_copy(dacc_v, dsum_out.at[wid])


def kernel(lsm, target):
    tgt = target.astype(jnp.int32)
    tc_total, tc_gsum = _tc_part(tgt, lsm)
    sc_dsum, sc_gsum = _sc_part(tgt, lsm)
    total = tc_total[0, 0] + jnp.sum(sc_dsum)
    gsum = tc_gsum[0, 0] + jnp.sum(sc_gsum)
    scale = jnp.float32(_CONF - _BASE)
    return -(jnp.float32(_BASE) * total + scale * gsum) / jnp.float32(_B)
